# baseline (speedup 1.0000x reference)
import functools

import jax
import jax.numpy as jnp
from jax import lax
from jax.experimental import pallas as pl
from jax.experimental.pallas import tpu as pltpu
from jax.experimental.pallas import tpu_sc as plsc

N_NODES = 10000
N_EDGES = 320000
D = 128

NC = 2
NS = 16
NW = NC * NS

N_PAD = 10240
ROWS_PER_TILE = N_PAD // NS

E_PER_TILE = N_EDGES // NW

DEG_CHUNK = 80
DEG_NCHUNK = E_PER_TILE // DEG_CHUNK

AGG_CHUNK = 50
AGG_NCHUNK = E_PER_TILE // AGG_CHUNK
GSZ = 4
NGROUP = AGG_NCHUNK // GSZ

_mesh = plsc.VectorSubcoreMesh(core_axis_name="c", subcore_axis_name="s")


@functools.partial(
    pl.kernel,
    mesh=_mesh,
    out_type=jax.ShapeDtypeStruct((NC, N_PAD), jnp.float32),
    scratch_types=[
        pltpu.VMEM((DEG_NCHUNK, DEG_CHUNK), jnp.int32),
        pltpu.VMEM((DEG_CHUNK,), jnp.float32),
        pltpu.VMEM((ROWS_PER_TILE,), jnp.float32),
        pltpu.VMEM_SHARED((N_PAD,), jnp.float32),
    ],
)
def _deg_kernel(dst_hbm, deg_out, dst_v, ones_v, zeros_v, acc):
    cid = lax.axis_index("c")
    sid = lax.axis_index("s")
    wid = cid * NS + sid

    for i in range(DEG_CHUNK // 16):
        ones_v[pl.ds(i * 16, 16)] = jnp.ones((16,), jnp.float32)
    for i in range(ROWS_PER_TILE // 16):
        zeros_v[pl.ds(i * 16, 16)] = jnp.zeros((16,), jnp.float32)

    pltpu.sync_copy(zeros_v, acc.at[pl.ds(sid * ROWS_PER_TILE, ROWS_PER_TILE)])
    pltpu.sync_copy(dst_hbm.at[wid], dst_v)
    plsc.subcore_barrier()

    def body(j, carry):
        pltpu.sync_copy(ones_v, acc.at[dst_v.at[j]], add=True)
        return carry

    lax.fori_loop(0, DEG_NCHUNK, body, 0)
    plsc.subcore_barrier()

    sl = pl.ds(sid * ROWS_PER_TILE, ROWS_PER_TILE)
    pltpu.sync_copy(acc.at[sl], deg_out.at[cid, sl])


@functools.partial(
    pl.kernel,
    mesh=_mesh,
    out_type=jax.ShapeDtypeStruct((NC, N_PAD, D), jnp.float32),
    scratch_types=[
        pltpu.VMEM((2, GSZ, AGG_CHUNK), jnp.int32),
        pltpu.VMEM((2, GSZ, AGG_CHUNK), jnp.int32),
        pltpu.VMEM((GSZ, AGG_CHUNK, D), jnp.float32),
        pltpu.VMEM((8, D), jnp.float32),
        pltpu.VMEM_SHARED((N_PAD, D), jnp.float32),
        pltpu.SemaphoreType.DMA,
        pltpu.SemaphoreType.DMA,
        pltpu.SemaphoreType.DMA,
        pltpu.SemaphoreType.DMA,
        pltpu.SemaphoreType.DMA,
        pltpu.SemaphoreType.DMA,
        pltpu.SemaphoreType.DMA,
        pltpu.SemaphoreType.DMA,
        pltpu.SemaphoreType.DMA,
        pltpu.SemaphoreType.DMA,
    ],
)
def _agg_kernel(src_hbm, dst_hbm, h2_hbm, agg_out,
                sring, dring, rows, ztile, acc,
                g0, g1, g2, g3, s0, s1, s2, s3, r0, r1):
    cid = lax.axis_index("c")
    sid = lax.axis_index("s")
    wid = cid * NS + sid
    gsem = (g0, g1, g2, g3)
    ssem = (s0, s1, s2, s3)
    rsem = (r0, r1)

    for r in range(8):
        for c in range(D // 16):
            ztile[r, pl.ds(c * 16, 16)] = jnp.zeros((16,), jnp.float32)

    def zcopy(j, carry):
        pltpu.sync_copy(
            ztile, acc.at[pl.ds(sid * ROWS_PER_TILE + j * 8, 8)])
        return carry
    lax.fori_loop(0, ROWS_PER_TILE // 8, zcopy, 0)
    plsc.subcore_barrier()

    def ring_issue(g, parity):
        sem = rsem[parity]
        sl = pl.ds(GSZ * g, GSZ)
        pltpu.async_copy(src_hbm.at[wid, sl], sring.at[parity], sem)
        pltpu.async_copy(dst_hbm.at[wid, sl], dring.at[parity], sem)

    def ring_wait(g, parity):
        sem = rsem[parity]
        sl = pl.ds(GSZ * g, GSZ)
        pltpu.make_async_copy(src_hbm.at[wid, sl], sring.at[parity], sem).wait()
        pltpu.make_async_copy(dst_hbm.at[wid, sl], dring.at[parity], sem).wait()

    def proc(g, parity, has_prev):
        ring_wait(g, parity)
        for k in range(GSZ):
            if has_prev is not None:
                if has_prev is True:
                    pltpu.make_async_copy(
                        rows.at[k], acc.at[dring.at[1 - parity, k]],
                        ssem[k]).wait()
                else:
                    @pl.when(has_prev)
                    def _():
                        pltpu.make_async_copy(
                            rows.at[k], acc.at[dring.at[1 - parity, k]],
                            ssem[k]).wait()
            pltpu.async_copy(
                h2_hbm.at[sring.at[parity, k]], rows.at[k], gsem[k])
        @pl.when(g + 1 < NGROUP)
        def _():
            ring_issue(g + 1, 1 - parity)
        for k in range(GSZ):
            pltpu.make_async_copy(
                h2_hbm.at[sring.at[parity, k]], rows.at[k], gsem[k]).wait()
            pltpu.async_copy(
                rows.at[k], acc.at[dring.at[parity, k]], ssem[k], add=True)

    ring_issue(0, 0)

    def body(t, carry):
        proc(2 * t, 0, has_prev=(t > 0))
        proc(2 * t + 1, 1, has_prev=True)
        return carry

    lax.fori_loop(0, NGROUP // 2, body, 0)

    for k in range(GSZ):
        pltpu.make_async_copy(
            rows.at[k], acc.at[dring.at[1, k]], ssem[k]).wait()
    plsc.subcore_barrier()

    sl = pl.ds(sid * ROWS_PER_TILE, ROWS_PER_TILE)
    pltpu.sync_copy(acc.at[sl], agg_out.at[cid, sl])


_BLK = 512
_GRID = N_PAD // _BLK


def _h2_body(x_ref, w_ref, deg_ref, h2_ref):
    deg = 1.0 + deg_ref[0, :] + deg_ref[1, :]
    dis = lax.rsqrt(deg)
    h = jnp.dot(x_ref[...], w_ref[...], preferred_element_type=jnp.float32)
    h2_ref[...] = h * dis[:, None]


def _h2_call(x_pad, W, degp):
    return pl.pallas_call(
        _h2_body,
        grid=(_GRID,),
        in_specs=[
            pl.BlockSpec((_BLK, D), lambda i: (i, 0)),
            pl.BlockSpec((D, D), lambda i: (0, 0)),
            pl.BlockSpec((NC, _BLK), lambda i: (0, i)),
        ],
        out_specs=pl.BlockSpec((_BLK, D), lambda i: (i, 0)),
        out_shape=jax.ShapeDtypeStruct((N_PAD, D), jnp.float32),
    )(x_pad, W, degp)


def _out_body(agg_ref, h2_ref, deg_ref, b_ref, out_ref):
    deg = 1.0 + deg_ref[0, :] + deg_ref[1, :]
    dis = lax.rsqrt(deg)
    s = agg_ref[0] + agg_ref[1] + h2_ref[...]
    out_ref[...] = jnp.maximum(s * dis[:, None] + b_ref[...], 0.0)


def _out_call(agg, h2, degp, b2):
    return pl.pallas_call(
        _out_body,
        grid=(_GRID,),
        in_specs=[
            pl.BlockSpec((NC, _BLK, D), lambda i: (0, i, 0)),
            pl.BlockSpec((_BLK, D), lambda i: (i, 0)),
            pl.BlockSpec((NC, _BLK), lambda i: (0, i)),
            pl.BlockSpec((1, D), lambda i: (0, 0)),
        ],
        out_specs=pl.BlockSpec((_BLK, D), lambda i: (i, 0)),
        out_shape=jax.ShapeDtypeStruct((N_PAD, D), jnp.float32),
    )(agg, h2, degp, b2)


def kernel(x, edge_index, W, b):
    src = edge_index[0].astype(jnp.int32)
    dst = edge_index[1].astype(jnp.int32)

    dst_deg = dst.reshape(NW, DEG_NCHUNK, DEG_CHUNK)
    src_agg = src.reshape(NW, AGG_NCHUNK, AGG_CHUNK)
    dst_agg = dst.reshape(NW, AGG_NCHUNK, AGG_CHUNK)

    degp = _deg_kernel(dst_deg)

    x_pad = jnp.pad(x, ((0, N_PAD - N_NODES), (0, 0)))
    h2 = _h2_call(x_pad, W, degp)

    agg = _agg_kernel(src_agg, dst_agg, h2)

    out = _out_call(agg, h2, degp, b.reshape(1, D))
    return out[:N_NODES]


# speedup vs baseline: 1.0970x; 1.0970x over previous
import functools

import jax
import jax.numpy as jnp
from jax import lax
from jax.experimental import pallas as pl
from jax.experimental.pallas import tpu as pltpu
from jax.experimental.pallas import tpu_sc as plsc

N_NODES = 10000
N_EDGES = 320000
D = 128

NC = 2
NS = 16
NW = NC * NS

N_PAD = 10240
ROWS_PER_TILE = N_PAD // NS

E_PER_TILE = N_EDGES // NW

DEG_CHUNK = 80
DEG_NCHUNK = E_PER_TILE // DEG_CHUNK

AGG_CHUNK = 125
AGG_NCHUNK = E_PER_TILE // AGG_CHUNK

_mesh = plsc.VectorSubcoreMesh(core_axis_name="c", subcore_axis_name="s")


@functools.partial(
    pl.kernel,
    mesh=_mesh,
    out_type=jax.ShapeDtypeStruct((NC, N_PAD), jnp.float32),
    scratch_types=[
        pltpu.VMEM((DEG_NCHUNK, DEG_CHUNK), jnp.int32),
        pltpu.VMEM((DEG_CHUNK,), jnp.float32),
        pltpu.VMEM((ROWS_PER_TILE,), jnp.float32),
        pltpu.VMEM_SHARED((N_PAD,), jnp.float32),
    ],
)
def _deg_kernel(dst_hbm, deg_out, dst_v, ones_v, zeros_v, acc):
    cid = lax.axis_index("c")
    sid = lax.axis_index("s")
    wid = cid * NS + sid

    for i in range(DEG_CHUNK // 16):
        ones_v[pl.ds(i * 16, 16)] = jnp.ones((16,), jnp.float32)
    for i in range(ROWS_PER_TILE // 16):
        zeros_v[pl.ds(i * 16, 16)] = jnp.zeros((16,), jnp.float32)

    pltpu.sync_copy(zeros_v, acc.at[pl.ds(sid * ROWS_PER_TILE, ROWS_PER_TILE)])
    pltpu.sync_copy(dst_hbm.at[wid], dst_v)
    plsc.subcore_barrier()

    def body(j, carry):
        pltpu.sync_copy(ones_v, acc.at[dst_v.at[j]], add=True)
        return carry

    lax.fori_loop(0, DEG_NCHUNK, body, 0)
    plsc.subcore_barrier()

    sl = pl.ds(sid * ROWS_PER_TILE, ROWS_PER_TILE)
    pltpu.sync_copy(acc.at[sl], deg_out.at[cid, sl])


@functools.partial(
    pl.kernel,
    mesh=_mesh,
    compiler_params=pltpu.CompilerParams(use_tc_tiling_on_sc=False),
    out_type=jax.ShapeDtypeStruct((NC, N_PAD, D), jnp.bfloat16),
    scratch_types=[
        pltpu.VMEM((AGG_NCHUNK, AGG_CHUNK), jnp.int32),
        pltpu.VMEM((AGG_NCHUNK, AGG_CHUNK), jnp.int32),
        pltpu.VMEM((AGG_CHUNK, D), jnp.bfloat16),
        pltpu.VMEM((AGG_CHUNK, D), jnp.bfloat16),
        pltpu.VMEM((8, D), jnp.bfloat16),
        pltpu.VMEM_SHARED((N_PAD, D), jnp.bfloat16),
        pltpu.SemaphoreType.DMA,
        pltpu.SemaphoreType.DMA,
    ],
)
def _agg_kernel(src_hbm, dst_hbm, h2_hbm, agg_out,
                src_v, dst_v, rows0, rows1, ztile, acc, sem0, sem1):
    cid = lax.axis_index("c")
    sid = lax.axis_index("s")
    wid = cid * NS + sid

    for r in range(8):
        for c in range(D // 32):
            ztile[r, pl.ds(c * 32, 32)] = jnp.zeros((32,), jnp.bfloat16)

    def zcopy(j, carry):
        pltpu.sync_copy(
            ztile, acc.at[pl.ds(sid * ROWS_PER_TILE + j * 8, 8)])
        return carry
    lax.fori_loop(0, ROWS_PER_TILE // 8, zcopy, 0)

    pltpu.sync_copy(src_hbm.at[wid], src_v)
    pltpu.sync_copy(dst_hbm.at[wid], dst_v)
    plsc.subcore_barrier()

    def gat(j, rows, sem):
        return pltpu.async_copy(h2_hbm.at[src_v.at[j]], rows, sem)

    gat(0, rows0, sem0)

    def body(g, carry):
        j0 = 2 * g
        j1 = j0 + 1
        j2 = j0 + 2
        gat(j1, rows1, sem1)
        pltpu.make_async_copy(h2_hbm.at[src_v.at[j0]], rows0, sem0).wait()
        pltpu.sync_copy(rows0, acc.at[dst_v.at[j0]], add=True)

        @pl.when(j2 < AGG_NCHUNK)
        def _():
            gat(j2, rows0, sem0)

        pltpu.make_async_copy(h2_hbm.at[src_v.at[j1]], rows1, sem1).wait()
        pltpu.sync_copy(rows1, acc.at[dst_v.at[j1]], add=True)
        return carry

    lax.fori_loop(0, AGG_NCHUNK // 2, body, 0)
    plsc.subcore_barrier()

    sl = pl.ds(sid * ROWS_PER_TILE, ROWS_PER_TILE)
    pltpu.sync_copy(acc.at[sl], agg_out.at[cid, sl])


_BLK = 512
_GRID = N_PAD // _BLK


def _h2_body(x_ref, w_ref, deg_ref, h2_ref, h2b_ref):
    deg = 1.0 + deg_ref[0, :] + deg_ref[1, :]
    dis = lax.rsqrt(deg)
    h = jnp.dot(x_ref[...], w_ref[...], preferred_element_type=jnp.float32)
    h2 = h * dis[:, None]
    h2_ref[...] = h2
    h2b_ref[...] = h2.astype(jnp.bfloat16)


def _h2_call(x_pad, W, degp):
    return pl.pallas_call(
        _h2_body,
        grid=(_GRID,),
        in_specs=[
            pl.BlockSpec((_BLK, D), lambda i: (i, 0)),
            pl.BlockSpec((D, D), lambda i: (0, 0)),
            pl.BlockSpec((NC, _BLK), lambda i: (0, i)),
        ],
        out_specs=[
            pl.BlockSpec((_BLK, D), lambda i: (i, 0)),
            pl.BlockSpec((_BLK, D), lambda i: (i, 0)),
        ],
        out_shape=[
            jax.ShapeDtypeStruct((N_PAD, D), jnp.float32),
            jax.ShapeDtypeStruct((N_PAD, D), jnp.bfloat16),
        ],
    )(x_pad, W, degp)


def _out_body(agg_ref, h2_ref, deg_ref, b_ref, out_ref):
    deg = 1.0 + deg_ref[0, :] + deg_ref[1, :]
    dis = lax.rsqrt(deg)
    s = (agg_ref[0].astype(jnp.float32) + agg_ref[1].astype(jnp.float32)
         + h2_ref[...])
    out_ref[...] = jnp.maximum(s * dis[:, None] + b_ref[...], 0.0)


def _out_call(agg, h2, degp, b2):
    return pl.pallas_call(
        _out_body,
        grid=(_GRID,),
        in_specs=[
            pl.BlockSpec((NC, _BLK, D), lambda i: (0, i, 0)),
            pl.BlockSpec((_BLK, D), lambda i: (i, 0)),
            pl.BlockSpec((NC, _BLK), lambda i: (0, i)),
            pl.BlockSpec((1, D), lambda i: (0, 0)),
        ],
        out_specs=pl.BlockSpec((_BLK, D), lambda i: (i, 0)),
        out_shape=jax.ShapeDtypeStruct((N_PAD, D), jnp.float32),
    )(agg, h2, degp, b2)


def kernel(x, edge_index, W, b):
    src = edge_index[0].astype(jnp.int32)
    dst = edge_index[1].astype(jnp.int32)

    dst_deg = dst.reshape(NW, DEG_NCHUNK, DEG_CHUNK)
    src_agg = src.reshape(NW, AGG_NCHUNK, AGG_CHUNK)
    dst_agg = dst.reshape(NW, AGG_NCHUNK, AGG_CHUNK)

    degp = _deg_kernel(dst_deg)

    x_pad = jnp.pad(x, ((0, N_PAD - N_NODES), (0, 0)))
    h2, h2b = _h2_call(x_pad, W, degp)

    agg = _agg_kernel(src_agg, dst_agg, h2b)

    out = _out_call(agg, h2, degp, b.reshape(1, D))
    return out[:N_NODES]
